# baseline (speedup 1.0000x reference)
import functools

import jax
import jax.numpy as jnp
from jax import lax
from jax.experimental import pallas as pl
from jax.experimental.pallas import tpu as pltpu
from jax.experimental.pallas import tpu_sc as plsc

D = 64
L = 16
NC, NS = 2, 16
NW = NC * NS
EPS = 1e-5


def _rsqrt(x):
    i = plsc.bitcast(x, jnp.int32)
    i = jnp.int32(0x5F3759DF) - lax.shift_right_logical(i, 1)
    y = plsc.bitcast(i, jnp.float32)
    h = x * jnp.float32(-0.5)
    for _ in range(2):
        y = y * (jnp.float32(1.5) + h * y * y)
    return y


def _bcast(s):
    return lax.broadcast_in_dim(s, (L,), ())


def _make_sc_kernel(batch, hist):
    chunk = hist
    bat_per_w = batch // NW
    per_w = bat_per_w * hist
    splits = []
    off = 0
    while off < chunk:
        n = min(128, chunk - off)
        splits.append((off, n))
        off += n
    assert batch % NW == 0 and all(o % 8 == 0 for o, _ in splits)

    mesh = plsc.VectorSubcoreMesh(core_axis_name="c", subcore_axis_name="s")

    @functools.partial(
        pl.kernel,
        out_type=jax.ShapeDtypeStruct((batch, hist, D), jnp.float32),
        mesh=mesh,
        compiler_params=pltpu.CompilerParams(
            needs_layout_passes=False, use_tc_tiling_on_sc=False),
        scratch_types=[
            pltpu.VMEM((per_w,), jnp.int32),
            pltpu.VMEM((chunk, 2 * D), jnp.float32),
            pltpu.VMEM((chunk, 2 * D), jnp.float32),
            pltpu.VMEM((chunk, D), jnp.float32),
            pltpu.VMEM((chunk, D), jnp.float32),
            pltpu.VMEM((D,), jnp.float32),
            pltpu.VMEM((D,), jnp.float32),
            pltpu.SemaphoreType.DMA,
            pltpu.SemaphoreType.DMA,
            pltpu.SemaphoreType.DMA,
            pltpu.SemaphoreType.DMA,
        ],
    )
    def sc_kernel(x_hbm, table_hbm, gamma_hbm, beta_hbm, out_hbm,
                  idx_v, rows0, rows1, outb0, outb1, gam_v, bet_v,
                  gsem0, gsem1, osem0, osem1):
        wid = lax.axis_index("s") * NC + lax.axis_index("c")
        rows = (rows0, rows1)
        outb = (outb0, outb1)
        gsem = (gsem0, gsem1)
        osem = (osem0, osem1)

        pltpu.sync_copy(gamma_hbm, gam_v)
        pltpu.sync_copy(beta_hbm, bet_v)
        pltpu.sync_copy(x_hbm.at[pl.ds(wid * per_w, per_w)], idx_v)

        gam = [gam_v[pl.ds(j * L, L)] for j in range(D // L)]
        bet = [bet_v[pl.ds(j * L, L)] for j in range(D // L)]

        def start_gather(g, b):
            for off, n in splits:
                pltpu.async_copy(
                    table_hbm.at[idx_v.at[pl.ds(g * chunk + off, n)]],
                    rows[b].at[pl.ds(off, n)],
                    gsem[b])

        def wait_gather(b):
            pltpu.make_async_copy(
                table_hbm.at[pl.ds(0, chunk)], rows[b], gsem[b]).wait()

        def start_scatter(g, b):
            pltpu.async_copy(
                outb[b], out_hbm.at[wid * bat_per_w + g], osem[b])

        def wait_scatter(b):
            pltpu.make_async_copy(outb[b], out_hbm.at[0], osem[b]).wait()

        def compute(b):
            rv, ov = rows[b], outb[b]

            @plsc.parallel_loop(0, chunk, unroll=4)
            def ln_row(r):
                v = [rv[r, pl.ds(j * L, L)] for j in range(D // L)]
                vs = (v[0] + v[1]) + (v[2] + v[3])
                vq = (v[0] * v[0] + v[1] * v[1]) + (v[2] * v[2] + v[3] * v[3])
                sv = _bcast(jnp.sum(vs))
                qv = _bcast(jnp.sum(vq))
                meanv = sv * jnp.float32(1.0 / D)
                varv = qv * jnp.float32(1.0 / D) - meanv * meanv
                rstd = _rsqrt(jnp.maximum(varv, jnp.float32(0.0))
                              + jnp.float32(EPS))
                for j in range(D // L):
                    ov[r, pl.ds(j * L, L)] = \
                        (v[j] - meanv) * (rstd * gam[j]) + bet[j]

        start_gather(0, 0)
        start_gather(1, 1)
        for g in (0, 1):
            wait_gather(g)
            compute(g)
            start_scatter(g, g)
            start_gather(g + 2, g)

        def pair_body(i, carry):
            for b in range(2):
                g = 2 * i + b
                wait_gather(b)
                wait_scatter(b)
                compute(b)
                start_scatter(g, b)
                start_gather(g + 2, b)
            return carry

        lax.fori_loop(1, bat_per_w // 2 - 1, pair_body, 0)

        for b in range(2):
            g = bat_per_w - 2 + b
            wait_gather(b)
            wait_scatter(b)
            compute(b)
            start_scatter(g, b)
        for b in range(2):
            wait_scatter(b)

    return sc_kernel


def kernel(x, table, gamma, beta):
    b, h = x.shape
    x1 = x.reshape(b * h).astype(jnp.int32)
    tlin = jnp.pad(table, ((0, 0), (0, D)))
    return _make_sc_kernel(b, h)(x1, tlin, gamma, beta)
